# baseline (speedup 1.0000x reference)
import functools

import jax
import jax.numpy as jnp
from jax import lax
from jax.experimental import pallas as pl
from jax.experimental.pallas import tpu as pltpu
from jax.experimental.pallas import tpu_sc as plsc

B, S, D, H, E, K = 1, 2048, 1024, 2048, 8, 2
N = B * S
NK = N * K
BM = 256
NB = N // BM
BM2 = 128
G_MAX = NK // BM2 + E
M_PAD = G_MAX * BM2

NC, NS = 2, 16
NW = NC * NS
A_PER_W = NK // NW
T_PER_W = N // NW
CH = 32
CC = 16


def _router_body(x_ref, rw_ref, i12_ref, w12_ref, rel_ref, stats_ref, loss_ref):
    nb = pl.program_id(0)
    xb = x_ref[...]
    logits = jax.lax.dot_general(
        xb, rw_ref[...], (((1,), (1,)), ((), ())),
        preferred_element_type=jnp.float32)
    m = jnp.max(logits, axis=-1, keepdims=True)
    p = jnp.exp(logits - m)
    probs = p / jnp.sum(p, axis=-1, keepdims=True)

    lane = jax.lax.broadcasted_iota(jnp.int32, (BM, E), 1)
    p1 = jnp.max(probs, axis=-1, keepdims=True)
    i1 = jnp.min(jnp.where(probs == p1, lane, E), axis=-1, keepdims=True)
    m1 = lane == i1
    probs2 = jnp.where(m1, -jnp.inf, probs)
    p2 = jnp.max(probs2, axis=-1, keepdims=True)
    i2 = jnp.min(jnp.where(probs2 == p2, lane, E), axis=-1, keepdims=True)
    m2 = lane == i2

    denom = p1 + p2 + 1e-8
    i12_ref[...] = jnp.concatenate([i1, i2], axis=1)
    w12_ref[...] = jnp.concatenate([p1 / denom, p2 / denom], axis=1)

    @pl.when(nb == 0)
    def _():
        stats_ref[...] = jnp.zeros_like(stats_ref)

    m1f = m1.astype(jnp.float32)
    m2f = m2.astype(jnp.float32)
    run0 = stats_ref[1:2, :]
    r_io = jax.lax.broadcasted_iota(jnp.int32, (BM, BM), 0)
    c_io = jax.lax.broadcasted_iota(jnp.int32, (BM, BM), 1)
    tril = (r_io > c_io).astype(jnp.float32)
    pref = jax.lax.dot_general(
        tril, m1f + m2f, (((1,), (0,)), ((), ())),
        preferred_element_type=jnp.float32) + run0
    rel0 = jnp.sum(m1f * pref, axis=-1, keepdims=True)
    rel1 = jnp.sum(m2f * pref, axis=-1, keepdims=True)
    rel_ref[...] = jnp.concatenate([rel0, rel1], axis=1).astype(jnp.int32)

    psum = jnp.sum(probs, axis=0, keepdims=True)
    csum = jnp.sum(m1f + m2f, axis=0, keepdims=True)
    stats_ref[...] += jnp.concatenate([psum, csum], axis=0)

    @pl.when(nb == NB - 1)
    def _():
        st = stats_ref[...]
        mean_probs = st[0:1, :] / N
        fracs = st[1:2, :] / (N * K)
        loss_ref[...] = jnp.sum(mean_probs * fracs, keepdims=True).reshape(1, 1) * E


def _router(x_flat, router_W):
    return pl.pallas_call(
        _router_body,
        grid=(NB,),
        in_specs=[
            pl.BlockSpec((BM, D), lambda nb: (nb, 0)),
            pl.BlockSpec((E, D), lambda nb: (0, 0)),
        ],
        out_specs=[
            pl.BlockSpec((BM, K), lambda nb: (nb, 0)),
            pl.BlockSpec((BM, K), lambda nb: (nb, 0)),
            pl.BlockSpec((BM, K), lambda nb: (nb, 0)),
            pl.BlockSpec((2, E), lambda nb: (0, 0)),
            pl.BlockSpec((1, 1), lambda nb: (0, 0)),
        ],
        out_shape=[
            jax.ShapeDtypeStruct((N, K), jnp.int32),
            jax.ShapeDtypeStruct((N, K), jnp.float32),
            jax.ShapeDtypeStruct((N, K), jnp.int32),
            jax.ShapeDtypeStruct((2, E), jnp.float32),
            jax.ShapeDtypeStruct((1, 1), jnp.float32),
        ],
    )(x_flat, router_W)


def _dispatch_body(x_hbm, inv_hbm, xg_hbm, tok_v, idx_v, rows_v, gsem, ssem):
    wid = lax.axis_index("s") * NC + lax.axis_index("c")
    a0 = wid * A_PER_W
    lanes = jax.lax.broadcasted_iota(jnp.int32, (16,), 0)
    for c in range(A_PER_W // CH):
        for j in range(CH // 16):
            a = a0 + c * CH + j * 16 + lanes
            tok_v[pl.ds(j * 16, 16)] = lax.rem(a, N)
        pltpu.sync_copy(inv_hbm.at[pl.ds(a0 + c * CH, CH)], idx_v)
        pltpu.async_copy(x_hbm.at[tok_v], rows_v, gsem).wait()
        pltpu.async_copy(rows_v, xg_hbm.at[idx_v], ssem).wait()


def _dispatch(x_flat, inv_flat):
    mesh = plsc.VectorSubcoreMesh(core_axis_name="c", subcore_axis_name="s")
    f = functools.partial(
        pl.kernel, mesh=mesh,
        out_type=jax.ShapeDtypeStruct((M_PAD, D), jnp.float32),
        scratch_types=[
            pltpu.VMEM((CH,), jnp.int32),
            pltpu.VMEM((CH,), jnp.int32),
            pltpu.VMEM((CH, D), jnp.float32),
            pltpu.SemaphoreType.DMA,
            pltpu.SemaphoreType.DMA,
        ],
    )(_dispatch_body)
    return f(x_flat, inv_flat)


def _combine_body(yg_hbm, inv_hbm, wrep_hbm, out_hbm,
                  idx0_v, idx1_v, w0_v, w1_v, buf0, buf1, sem0, sem1):
    wid = lax.axis_index("s") * NC + lax.axis_index("c")
    tb = wid * T_PER_W

    def chunk_step(c, _):
        tc0 = tb + c * CC
        pltpu.sync_copy(inv_hbm.at[pl.ds(tc0, CC)], idx0_v)
        pltpu.sync_copy(inv_hbm.at[pl.ds(N + tc0, CC)], idx1_v)
        cp0 = pltpu.async_copy(yg_hbm.at[idx0_v], buf0, sem0)
        cp1 = pltpu.async_copy(yg_hbm.at[idx1_v], buf1, sem1)
        pltpu.sync_copy(wrep_hbm.at[0, pl.ds(tc0, CC), :], w0_v)
        pltpu.sync_copy(wrep_hbm.at[1, pl.ds(tc0, CC), :], w1_v)
        cp0.wait()
        cp1.wait()
        for r in range(CC):
            w0 = w0_v[r, :]
            w1 = w1_v[r, :]
            for i in range(D // 16):
                sl = pl.ds(i * 16, 16)
                buf0[r, sl] = buf0[r, sl] * w0 + buf1[r, sl] * w1
        pltpu.sync_copy(buf0, out_hbm.at[pl.ds(tc0, CC), :])
        return 0

    lax.fori_loop(0, T_PER_W // CC, chunk_step, 0)


def _combine(yg, inv_flat, wrep):
    mesh = plsc.VectorSubcoreMesh(core_axis_name="c", subcore_axis_name="s")
    f = functools.partial(
        pl.kernel, mesh=mesh,
        out_type=jax.ShapeDtypeStruct((N, D), jnp.float32),
        scratch_types=[
            pltpu.VMEM((CC,), jnp.int32),
            pltpu.VMEM((CC,), jnp.int32),
            pltpu.VMEM((CC, 16), jnp.float32),
            pltpu.VMEM((CC, 16), jnp.float32),
            pltpu.VMEM((CC, D), jnp.float32),
            pltpu.VMEM((CC, D), jnp.float32),
            pltpu.SemaphoreType.DMA,
            pltpu.SemaphoreType.DMA,
        ],
    )(_combine_body)
    return f(yg, inv_flat, wrep)


def _ffn_body(eid_ref, xg_ref, w1_ref, b1_ref, w2_ref, b2_ref, out_ref):
    xb = xg_ref[...]
    h = jax.lax.dot_general(
        xb, w1_ref[0], (((1,), (1,)), ((), ())),
        preferred_element_type=jnp.float32) + b1_ref[0]
    h = 0.5 * h * (1.0 + jax.lax.erf(h * 0.7071067811865476))
    out_ref[...] = jax.lax.dot_general(
        h, w2_ref[0], (((1,), (1,)), ((), ())),
        preferred_element_type=jnp.float32) + b2_ref[0]


def _ffn_grouped(xg, fc1_w, fc1_b, fc2_w, fc2_b, eid):
    grid_spec = pltpu.PrefetchScalarGridSpec(
        num_scalar_prefetch=1,
        grid=(G_MAX,),
        in_specs=[
            pl.BlockSpec((BM2, D), lambda g, eid_ref: (g, 0)),
            pl.BlockSpec((1, H, D), lambda g, eid_ref: (eid_ref[g], 0, 0)),
            pl.BlockSpec((1, 1, H), lambda g, eid_ref: (eid_ref[g], 0, 0)),
            pl.BlockSpec((1, D, H), lambda g, eid_ref: (eid_ref[g], 0, 0)),
            pl.BlockSpec((1, 1, D), lambda g, eid_ref: (eid_ref[g], 0, 0)),
        ],
        out_specs=pl.BlockSpec((BM2, D), lambda g, eid_ref: (g, 0)),
    )
    return pl.pallas_call(
        _ffn_body,
        grid_spec=grid_spec,
        out_shape=jax.ShapeDtypeStruct((M_PAD, D), jnp.float32),
    )(eid, xg, fc1_w, fc1_b.reshape(E, 1, H), fc2_w, fc2_b.reshape(E, 1, D))


def kernel(x, router_W, fc1_w, fc1_b, fc2_w, fc2_b, is_training):
    x_flat = x.reshape(N, D)
    i12, w12, rel, stats, loss = _router(x_flat, router_W)

    counts = stats[1].astype(jnp.int32)
    pc = ((counts + BM2 - 1) // BM2) * BM2
    pend = jnp.cumsum(pc)
    poff = pend - pc
    inv = (jnp.take(poff, i12, axis=0) + rel).T.reshape(NK)
    gstart = jnp.arange(G_MAX, dtype=jnp.int32) * BM2
    eid = jnp.minimum(jnp.sum((pend[None, :] <= gstart[:, None]).astype(jnp.int32),
                              axis=1), E - 1).astype(jnp.int32)
    wrep = jnp.broadcast_to(w12.T[:, :, None], (K, N, 16))

    xg = _dispatch(x_flat, inv)
    yg = _ffn_grouped(xg, fc1_w, fc1_b, fc2_w, fc2_b, eid)
    out_flat = _combine(yg, inv, wrep)
    return out_flat.reshape(x.shape), loss.reshape(())


# speedup vs baseline: 1.3505x; 1.3505x over previous
import functools

import jax
import jax.numpy as jnp
from jax import lax
from jax.experimental import pallas as pl
from jax.experimental.pallas import tpu as pltpu
from jax.experimental.pallas import tpu_sc as plsc

B, S, D, H, E, K = 1, 2048, 1024, 2048, 8, 2
N = B * S
NK = N * K
BM = 256
NB = N // BM
BM2 = 256
G_MAX = NK // BM2 + E
M_PAD = G_MAX * BM2

NC, NS = 2, 16
NW = NC * NS
A_PER_W = NK // NW
T_PER_W = N // NW
CH = 32
CC = 16


def _router_body(x_ref, rw_ref, i12_ref, w12_ref, rel_ref, stats_ref, loss_ref):
    nb = pl.program_id(0)
    xb = x_ref[...]
    logits = jax.lax.dot_general(
        xb, rw_ref[...], (((1,), (1,)), ((), ())),
        preferred_element_type=jnp.float32)
    m = jnp.max(logits, axis=-1, keepdims=True)
    p = jnp.exp(logits - m)
    probs = p / jnp.sum(p, axis=-1, keepdims=True)

    lane = jax.lax.broadcasted_iota(jnp.int32, (BM, E), 1)
    p1 = jnp.max(probs, axis=-1, keepdims=True)
    i1 = jnp.min(jnp.where(probs == p1, lane, E), axis=-1, keepdims=True)
    m1 = lane == i1
    probs2 = jnp.where(m1, -jnp.inf, probs)
    p2 = jnp.max(probs2, axis=-1, keepdims=True)
    i2 = jnp.min(jnp.where(probs2 == p2, lane, E), axis=-1, keepdims=True)
    m2 = lane == i2

    denom = p1 + p2 + 1e-8
    i12_ref[...] = jnp.concatenate([i1, i2], axis=1)
    w12_ref[...] = jnp.concatenate([p1 / denom, p2 / denom], axis=1)

    @pl.when(nb == 0)
    def _():
        stats_ref[...] = jnp.zeros_like(stats_ref)

    m1f = m1.astype(jnp.float32)
    m2f = m2.astype(jnp.float32)
    run0 = stats_ref[1:2, :]
    r_io = jax.lax.broadcasted_iota(jnp.int32, (BM, BM), 0)
    c_io = jax.lax.broadcasted_iota(jnp.int32, (BM, BM), 1)
    tril = (r_io > c_io).astype(jnp.float32)
    pref = jax.lax.dot_general(
        tril, m1f + m2f, (((1,), (0,)), ((), ())),
        preferred_element_type=jnp.float32) + run0
    rel0 = jnp.sum(m1f * pref, axis=-1, keepdims=True)
    rel1 = jnp.sum(m2f * pref, axis=-1, keepdims=True)
    rel_ref[...] = jnp.concatenate([rel0, rel1], axis=1).astype(jnp.int32)

    psum = jnp.sum(probs, axis=0, keepdims=True)
    csum = jnp.sum(m1f + m2f, axis=0, keepdims=True)
    stats_ref[...] += jnp.concatenate([psum, csum], axis=0)

    @pl.when(nb == NB - 1)
    def _():
        st = stats_ref[...]
        mean_probs = st[0:1, :] / N
        fracs = st[1:2, :] / (N * K)
        loss_ref[...] = jnp.sum(mean_probs * fracs, keepdims=True).reshape(1, 1) * E


def _router(x_flat, router_W):
    return pl.pallas_call(
        _router_body,
        grid=(NB,),
        in_specs=[
            pl.BlockSpec((BM, D), lambda nb: (nb, 0)),
            pl.BlockSpec((E, D), lambda nb: (0, 0)),
        ],
        out_specs=[
            pl.BlockSpec((BM, K), lambda nb: (nb, 0)),
            pl.BlockSpec((BM, K), lambda nb: (nb, 0)),
            pl.BlockSpec((BM, K), lambda nb: (nb, 0)),
            pl.BlockSpec((2, E), lambda nb: (0, 0)),
            pl.BlockSpec((1, 1), lambda nb: (0, 0)),
        ],
        out_shape=[
            jax.ShapeDtypeStruct((N, K), jnp.int32),
            jax.ShapeDtypeStruct((N, K), jnp.float32),
            jax.ShapeDtypeStruct((N, K), jnp.int32),
            jax.ShapeDtypeStruct((2, E), jnp.float32),
            jax.ShapeDtypeStruct((1, 1), jnp.float32),
        ],
    )(x_flat, router_W)


def _dispatch_body(x_hbm, inv3_hbm, xg_hbm, tok_v, pidx_v, rows_v, gsem, ssem):
    wid = lax.axis_index("s") * NC + lax.axis_index("c")
    a0 = wid * A_PER_W
    lanes = jax.lax.broadcasted_iota(jnp.int32, (16,), 0)
    pltpu.sync_copy(inv3_hbm.at[wid], pidx_v)
    for c in range(A_PER_W // CH):
        for j in range(CH // 16):
            a = a0 + c * CH + j * 16 + lanes
            tok_v[pl.ds(j * 16, 16)] = lax.rem(a, N)
        pltpu.async_copy(x_hbm.at[tok_v], rows_v, gsem).wait()
        pltpu.async_copy(rows_v, xg_hbm.at[pidx_v.at[c]], ssem).wait()


def _dispatch(x_flat, inv3):
    mesh = plsc.VectorSubcoreMesh(core_axis_name="c", subcore_axis_name="s")
    f = functools.partial(
        pl.kernel, mesh=mesh,
        out_type=jax.ShapeDtypeStruct((M_PAD, D), jnp.float32),
        scratch_types=[
            pltpu.VMEM((CH,), jnp.int32),
            pltpu.VMEM((A_PER_W // CH, CH), jnp.int32),
            pltpu.VMEM((CH, D), jnp.float32),
            pltpu.SemaphoreType.DMA,
            pltpu.SemaphoreType.DMA,
        ],
    )(_dispatch_body)
    return f(x_flat, inv3)


def _combine_body(yg_hbm, inv_hbm, wrep_hbm, out_hbm,
                  idx0_v, idx1_v, w0_v, w1_v, buf0, buf1, sem0, sem1):
    wid = lax.axis_index("s") * NC + lax.axis_index("c")
    tb = wid * T_PER_W
    pltpu.sync_copy(inv_hbm.at[pl.ds(tb, T_PER_W)], idx0_v)
    pltpu.sync_copy(inv_hbm.at[pl.ds(N + tb, T_PER_W)], idx1_v)
    pltpu.sync_copy(wrep_hbm.at[0, pl.ds(tb, T_PER_W), :], w0_v)
    pltpu.sync_copy(wrep_hbm.at[1, pl.ds(tb, T_PER_W), :], w1_v)

    def chunk_step(c, _):
        tc0 = tb + c * CC
        cp0 = pltpu.async_copy(yg_hbm.at[idx0_v.at[pl.ds(c * CC, CC)]],
                               buf0, sem0)
        cp1 = pltpu.async_copy(yg_hbm.at[idx1_v.at[pl.ds(c * CC, CC)]],
                               buf1, sem1)
        cp0.wait()
        cp1.wait()
        for r in range(CC):
            w0 = w0_v[c * CC + r, :]
            w1 = w1_v[c * CC + r, :]
            for i in range(D // 16):
                sl = pl.ds(i * 16, 16)
                buf0[r, sl] = buf0[r, sl] * w0 + buf1[r, sl] * w1
        pltpu.sync_copy(buf0, out_hbm.at[pl.ds(tc0, CC), :])
        return 0

    lax.fori_loop(0, T_PER_W // CC, chunk_step, 0)


def _combine(yg, inv_flat, wrep):
    mesh = plsc.VectorSubcoreMesh(core_axis_name="c", subcore_axis_name="s")
    f = functools.partial(
        pl.kernel, mesh=mesh,
        out_type=jax.ShapeDtypeStruct((N, D), jnp.float32),
        scratch_types=[
            pltpu.VMEM((T_PER_W,), jnp.int32),
            pltpu.VMEM((T_PER_W,), jnp.int32),
            pltpu.VMEM((T_PER_W, 16), jnp.float32),
            pltpu.VMEM((T_PER_W, 16), jnp.float32),
            pltpu.VMEM((CC, D), jnp.float32),
            pltpu.VMEM((CC, D), jnp.float32),
            pltpu.SemaphoreType.DMA,
            pltpu.SemaphoreType.DMA,
        ],
    )(_combine_body)
    return f(yg, inv_flat, wrep)


def _ffn_body(eid_ref, xg_ref, w1_ref, b1_ref, w2_ref, b2_ref, out_ref):
    xb = xg_ref[...]
    h = jax.lax.dot_general(
        xb, w1_ref[0], (((1,), (1,)), ((), ())),
        preferred_element_type=jnp.float32) + b1_ref[0]
    h = 0.5 * h * (1.0 + jax.lax.erf(h * 0.7071067811865476))
    out_ref[...] = jax.lax.dot_general(
        h, w2_ref[0], (((1,), (1,)), ((), ())),
        preferred_element_type=jnp.float32) + b2_ref[0]


def _ffn_grouped(xg, fc1_w, fc1_b, fc2_w, fc2_b, eid):
    grid_spec = pltpu.PrefetchScalarGridSpec(
        num_scalar_prefetch=1,
        grid=(G_MAX,),
        in_specs=[
            pl.BlockSpec((BM2, D), lambda g, eid_ref: (g, 0)),
            pl.BlockSpec((1, H, D), lambda g, eid_ref: (eid_ref[g], 0, 0)),
            pl.BlockSpec((1, 1, H), lambda g, eid_ref: (eid_ref[g], 0, 0)),
            pl.BlockSpec((1, D, H), lambda g, eid_ref: (eid_ref[g], 0, 0)),
            pl.BlockSpec((1, 1, D), lambda g, eid_ref: (eid_ref[g], 0, 0)),
        ],
        out_specs=pl.BlockSpec((BM2, D), lambda g, eid_ref: (g, 0)),
    )
    return pl.pallas_call(
        _ffn_body,
        grid_spec=grid_spec,
        out_shape=jax.ShapeDtypeStruct((M_PAD, D), jnp.float32),
    )(eid, xg, fc1_w, fc1_b.reshape(E, 1, H), fc2_w, fc2_b.reshape(E, 1, D))


def kernel(x, router_W, fc1_w, fc1_b, fc2_w, fc2_b, is_training):
    x_flat = x.reshape(N, D)
    i12, w12, rel, stats, loss = _router(x_flat, router_W)

    counts = stats[1].astype(jnp.int32)
    pc = ((counts + BM2 - 1) // BM2) * BM2
    pend = jnp.cumsum(pc)
    poff = pend - pc
    inv = (jnp.take(poff, i12, axis=0) + rel).T.reshape(NK)
    gstart = jnp.arange(G_MAX, dtype=jnp.int32) * BM2
    eid = jnp.minimum(jnp.sum((pend[None, :] <= gstart[:, None]).astype(jnp.int32),
                              axis=1), E - 1).astype(jnp.int32)
    wrep = jnp.broadcast_to(w12.T[:, :, None], (K, N, 16))

    xg = _dispatch(x_flat, inv.reshape(NW, A_PER_W // CH, CH))
    yg = _ffn_grouped(xg, fc1_w, fc1_b, fc2_w, fc2_b, eid)
    out_flat = _combine(yg, inv, wrep)
    return out_flat.reshape(x.shape), loss.reshape(())


# speedup vs baseline: 1.3537x; 1.0024x over previous
import functools

import jax
import jax.numpy as jnp
from jax import lax
from jax.experimental import pallas as pl
from jax.experimental.pallas import tpu as pltpu
from jax.experimental.pallas import tpu_sc as plsc

B, S, D, H, E, K = 1, 2048, 1024, 2048, 8, 2
N = B * S
NK = N * K
BM = 256
NB = N // BM
BM2 = 256
G_MAX = NK // BM2 + E
M_PAD = G_MAX * BM2

NC, NS = 2, 16
NW = NC * NS
A_PER_W = NK // NW
T_PER_W = N // NW
CH = 32
CC = 16


def _router_body(x_ref, rw_ref, i12_ref, w12_ref, rel_ref, stats_ref, loss_ref):
    nb = pl.program_id(0)
    xb = x_ref[...]
    logits = jax.lax.dot_general(
        xb, rw_ref[...], (((1,), (1,)), ((), ())),
        preferred_element_type=jnp.float32)
    m = jnp.max(logits, axis=-1, keepdims=True)
    p = jnp.exp(logits - m)
    probs = p / jnp.sum(p, axis=-1, keepdims=True)

    lane = jax.lax.broadcasted_iota(jnp.int32, (BM, E), 1)
    p1 = jnp.max(probs, axis=-1, keepdims=True)
    i1 = jnp.min(jnp.where(probs == p1, lane, E), axis=-1, keepdims=True)
    m1 = lane == i1
    probs2 = jnp.where(m1, -jnp.inf, probs)
    p2 = jnp.max(probs2, axis=-1, keepdims=True)
    i2 = jnp.min(jnp.where(probs2 == p2, lane, E), axis=-1, keepdims=True)
    m2 = lane == i2

    denom = p1 + p2 + 1e-8
    i12_ref[...] = jnp.concatenate([i1, i2], axis=1)
    w12_ref[...] = jnp.concatenate([p1 / denom, p2 / denom], axis=1)

    @pl.when(nb == 0)
    def _():
        stats_ref[...] = jnp.zeros_like(stats_ref)

    m1f = m1.astype(jnp.float32)
    m2f = m2.astype(jnp.float32)
    run0 = stats_ref[1:2, :]
    r_io = jax.lax.broadcasted_iota(jnp.int32, (BM, BM), 0)
    c_io = jax.lax.broadcasted_iota(jnp.int32, (BM, BM), 1)
    tril = (r_io > c_io).astype(jnp.float32)
    pref = jax.lax.dot_general(
        tril, m1f + m2f, (((1,), (0,)), ((), ())),
        preferred_element_type=jnp.float32) + run0
    rel0 = jnp.sum(m1f * pref, axis=-1, keepdims=True)
    rel1 = jnp.sum(m2f * pref, axis=-1, keepdims=True)
    rel_ref[...] = jnp.concatenate([rel0, rel1], axis=1).astype(jnp.int32)

    psum = jnp.sum(probs, axis=0, keepdims=True)
    csum = jnp.sum(m1f + m2f, axis=0, keepdims=True)
    stats_ref[...] += jnp.concatenate([psum, csum], axis=0)

    @pl.when(nb == NB - 1)
    def _():
        st = stats_ref[...]
        mean_probs = st[0:1, :] / N
        fracs = st[1:2, :] / (N * K)
        loss_ref[...] = jnp.sum(mean_probs * fracs, keepdims=True).reshape(1, 1) * E


def _router(x_flat, router_W):
    return pl.pallas_call(
        _router_body,
        grid=(NB,),
        in_specs=[
            pl.BlockSpec((BM, D), lambda nb: (nb, 0)),
            pl.BlockSpec((E, D), lambda nb: (0, 0)),
        ],
        out_specs=[
            pl.BlockSpec((BM, K), lambda nb: (nb, 0)),
            pl.BlockSpec((BM, K), lambda nb: (nb, 0)),
            pl.BlockSpec((BM, K), lambda nb: (nb, 0)),
            pl.BlockSpec((2, E), lambda nb: (0, 0)),
            pl.BlockSpec((1, 1), lambda nb: (0, 0)),
        ],
        out_shape=[
            jax.ShapeDtypeStruct((N, K), jnp.int32),
            jax.ShapeDtypeStruct((N, K), jnp.float32),
            jax.ShapeDtypeStruct((N, K), jnp.int32),
            jax.ShapeDtypeStruct((2, E), jnp.float32),
            jax.ShapeDtypeStruct((1, 1), jnp.float32),
        ],
    )(x_flat, router_W)


def _dispatch_body(x_hbm, inv3_hbm, xg_hbm, tok_v, pidx_v,
                   rows_a, rows_b, gsem, ssem_a, ssem_b):
    wid = lax.axis_index("s") * NC + lax.axis_index("c")
    a0 = wid * A_PER_W
    lanes = jax.lax.broadcasted_iota(jnp.int32, (16,), 0)
    pltpu.sync_copy(inv3_hbm.at[wid], pidx_v)
    bufs = (rows_a, rows_b)
    ssems = (ssem_a, ssem_b)
    scat = [None, None]
    for c in range(A_PER_W // CH):
        rows_v = bufs[c % 2]
        for j in range(CH // 16):
            a = a0 + c * CH + j * 16 + lanes
            tok_v[c, pl.ds(j * 16, 16)] = lax.rem(a, N)
        if scat[c % 2] is not None:
            scat[c % 2].wait()
        pltpu.async_copy(x_hbm.at[tok_v.at[c]], rows_v, gsem).wait()
        scat[c % 2] = pltpu.async_copy(rows_v, xg_hbm.at[pidx_v.at[c]],
                                       ssems[c % 2])
    scat[0].wait()
    scat[1].wait()


def _dispatch(x_flat, inv3):
    mesh = plsc.VectorSubcoreMesh(core_axis_name="c", subcore_axis_name="s")
    f = functools.partial(
        pl.kernel, mesh=mesh,
        out_type=jax.ShapeDtypeStruct((M_PAD, D), jnp.float32),
        scratch_types=[
            pltpu.VMEM((A_PER_W // CH, CH), jnp.int32),
            pltpu.VMEM((A_PER_W // CH, CH), jnp.int32),
            pltpu.VMEM((CH, D), jnp.float32),
            pltpu.VMEM((CH, D), jnp.float32),
            pltpu.SemaphoreType.DMA,
            pltpu.SemaphoreType.DMA,
            pltpu.SemaphoreType.DMA,
        ],
    )(_dispatch_body)
    return f(x_flat, inv3)


def _combine_body(yg_hbm, inv_hbm, wrep_hbm, out_hbm,
                  idx0_v, idx1_v, w0_v, w1_v, buf0, buf1, sem0, sem1):
    wid = lax.axis_index("s") * NC + lax.axis_index("c")
    tb = wid * T_PER_W
    pltpu.sync_copy(inv_hbm.at[pl.ds(tb, T_PER_W)], idx0_v)
    pltpu.sync_copy(inv_hbm.at[pl.ds(N + tb, T_PER_W)], idx1_v)
    pltpu.sync_copy(wrep_hbm.at[0, pl.ds(tb, T_PER_W), :], w0_v)
    pltpu.sync_copy(wrep_hbm.at[1, pl.ds(tb, T_PER_W), :], w1_v)

    def chunk_step(c, _):
        tc0 = tb + c * CC
        cp0 = pltpu.async_copy(yg_hbm.at[idx0_v.at[pl.ds(c * CC, CC)]],
                               buf0, sem0)
        cp1 = pltpu.async_copy(yg_hbm.at[idx1_v.at[pl.ds(c * CC, CC)]],
                               buf1, sem1)
        cp0.wait()
        cp1.wait()
        for r in range(CC):
            w0 = w0_v[c * CC + r, :]
            w1 = w1_v[c * CC + r, :]
            for i in range(D // 16):
                sl = pl.ds(i * 16, 16)
                buf0[r, sl] = buf0[r, sl] * w0 + buf1[r, sl] * w1
        pltpu.sync_copy(buf0, out_hbm.at[pl.ds(tc0, CC), :])
        return 0

    lax.fori_loop(0, T_PER_W // CC, chunk_step, 0)


def _combine(yg, inv_flat, wrep):
    mesh = plsc.VectorSubcoreMesh(core_axis_name="c", subcore_axis_name="s")
    f = functools.partial(
        pl.kernel, mesh=mesh,
        out_type=jax.ShapeDtypeStruct((N, D), jnp.float32),
        scratch_types=[
            pltpu.VMEM((T_PER_W,), jnp.int32),
            pltpu.VMEM((T_PER_W,), jnp.int32),
            pltpu.VMEM((T_PER_W, 16), jnp.float32),
            pltpu.VMEM((T_PER_W, 16), jnp.float32),
            pltpu.VMEM((CC, D), jnp.float32),
            pltpu.VMEM((CC, D), jnp.float32),
            pltpu.SemaphoreType.DMA,
            pltpu.SemaphoreType.DMA,
        ],
    )(_combine_body)
    return f(yg, inv_flat, wrep)


def _ffn_body(eid_ref, xg_ref, w1_ref, b1_ref, w2_ref, b2_ref, out_ref):
    xb = xg_ref[...]
    h = jax.lax.dot_general(
        xb, w1_ref[0], (((1,), (1,)), ((), ())),
        preferred_element_type=jnp.float32) + b1_ref[0]
    h = 0.5 * h * (1.0 + jax.lax.erf(h * 0.7071067811865476))
    out_ref[...] = jax.lax.dot_general(
        h, w2_ref[0], (((1,), (1,)), ((), ())),
        preferred_element_type=jnp.float32) + b2_ref[0]


def _ffn_grouped(xg, fc1_w, fc1_b, fc2_w, fc2_b, eid):
    grid_spec = pltpu.PrefetchScalarGridSpec(
        num_scalar_prefetch=1,
        grid=(G_MAX,),
        in_specs=[
            pl.BlockSpec((BM2, D), lambda g, eid_ref: (g, 0)),
            pl.BlockSpec((1, H, D), lambda g, eid_ref: (eid_ref[g], 0, 0)),
            pl.BlockSpec((1, 1, H), lambda g, eid_ref: (eid_ref[g], 0, 0)),
            pl.BlockSpec((1, D, H), lambda g, eid_ref: (eid_ref[g], 0, 0)),
            pl.BlockSpec((1, 1, D), lambda g, eid_ref: (eid_ref[g], 0, 0)),
        ],
        out_specs=pl.BlockSpec((BM2, D), lambda g, eid_ref: (g, 0)),
    )
    return pl.pallas_call(
        _ffn_body,
        grid_spec=grid_spec,
        out_shape=jax.ShapeDtypeStruct((M_PAD, D), jnp.float32),
    )(eid, xg, fc1_w, fc1_b.reshape(E, 1, H), fc2_w, fc2_b.reshape(E, 1, D))


def kernel(x, router_W, fc1_w, fc1_b, fc2_w, fc2_b, is_training):
    x_flat = x.reshape(N, D)
    i12, w12, rel, stats, loss = _router(x_flat, router_W)

    counts = stats[1].astype(jnp.int32)
    pc = ((counts + BM2 - 1) // BM2) * BM2
    pend = jnp.cumsum(pc)
    poff = pend - pc
    inv = (jnp.take(poff, i12, axis=0) + rel).T.reshape(NK)
    gstart = jnp.arange(G_MAX, dtype=jnp.int32) * BM2
    eid = jnp.minimum(jnp.sum((pend[None, :] <= gstart[:, None]).astype(jnp.int32),
                              axis=1), E - 1).astype(jnp.int32)
    wrep = jnp.broadcast_to(w12.T[:, :, None], (K, N, 16))

    xg = _dispatch(x_flat, inv.reshape(NW, A_PER_W // CH, CH))
    yg = _ffn_grouped(xg, fc1_w, fc1_b, fc2_w, fc2_b, eid)
    out_flat = _combine(yg, inv, wrep)
    return out_flat.reshape(x.shape), loss.reshape(())
